# baseline (speedup 1.0000x reference)
import functools

import jax
import jax.numpy as jnp
from jax.experimental import pallas as pl
from jax.experimental.pallas import tpu as pltpu


def _cbam_body(x_ref, w1_ref, w2_ref, wc_ref, o_ref, *,
               inv_hw, inv_c, height, width, ksize):
    NB = x_ref.shape[0]
    H, W, k = height, width, ksize
    HW = H * W
    p = (k - 1) // 2
    f32 = jnp.float32

    xs = [x_ref[nb].astype(f32) for nb in range(NB)]
    cols = [jnp.sum(xc, axis=-1, keepdims=True) * inv_hw for xc in xs]
    cols += [jnp.max(xc, axis=-1, keepdims=True) for xc in xs]
    pooled = jnp.concatenate(cols, axis=1)
    h = jnp.dot(w1_ref[...], pooled, preferred_element_type=f32)
    h = jnp.maximum(h, 0.0)
    g = jnp.dot(w2_ref[...], h, preferred_element_type=f32)
    cgate = jax.nn.sigmoid(g[:, :NB] + g[:, NB:])

    x1s = []
    mean_rows = []
    max_rows = []
    for nb in range(NB):
        x1 = xs[nb] * cgate[:, nb:nb + 1]
        x1s.append(x1)
        mean_rows.append(jnp.sum(x1, axis=0, keepdims=True) * inv_c)
        max_rows.append(jnp.max(x1, axis=0, keepdims=True))
    m2 = jnp.concatenate(mean_rows + max_rows, axis=0)

    lane = jax.lax.broadcasted_iota(jnp.int32, (1, HW), 1)
    orow = lane // W
    ocol = lane % W
    rowm = [((orow + (ki - p) >= 0) & (orow + (ki - p) < H)).astype(f32)
            for ki in range(k)]
    colm = [((ocol + (kj - p) >= 0) & (ocol + (kj - p) < W)).astype(f32)
            for kj in range(k)]
    accs = [jnp.zeros((2 * NB, HW), f32) for _ in range(2)]
    for ki in range(k):
        for kj in range(k):
            t = ki * k + kj
            s = (ki - p) * W + (kj - p)
            rolled = pltpu.roll(m2, (-s) % HW, axis=1)
            wv = wc_ref[:, t:t + 1]
            accs[t % 2] = accs[t % 2] + (rolled * wv) * (rowm[ki] * colm[kj])
    conv = accs[0] + accs[1]
    sgate = jax.nn.sigmoid(conv[:NB] + conv[NB:])

    out_dt = o_ref.dtype
    for nb in range(NB):
        o_ref[nb] = (x1s[nb] * sgate[nb:nb + 1, :]).astype(out_dt)


def kernel(x, w_fc1, w_fc2, w_sp):
    N, C, H, W = x.shape
    Cr = w_fc1.shape[0]
    k = w_sp.shape[-1]
    HW = H * W

    NB = 2 if N % 2 == 0 else 1

    x_flat = x.reshape(N, C, HW)
    wflat = w_sp.astype(jnp.float32).reshape(2, k * k)
    wcomb = jnp.concatenate(
        [jnp.tile(wflat[0:1], (NB, 1)), jnp.tile(wflat[1:2], (NB, 1))], axis=0)

    body = functools.partial(_cbam_body, inv_hw=1.0 / HW, inv_c=1.0 / C,
                             height=H, width=W, ksize=k)
    out_flat = pl.pallas_call(
        body,
        out_shape=jax.ShapeDtypeStruct((N, C, HW), x.dtype),
        grid=(N // NB,),
        in_specs=[
            pl.BlockSpec((NB, C, HW), lambda b: (b, 0, 0)),
            pl.BlockSpec((Cr, C), lambda b: (0, 0)),
            pl.BlockSpec((C, Cr), lambda b: (0, 0)),
            pl.BlockSpec((2 * NB, k * k), lambda b: (0, 0)),
        ],
        out_specs=pl.BlockSpec((NB, C, HW), lambda b: (b, 0, 0)),
        compiler_params=pltpu.CompilerParams(
            dimension_semantics=("parallel",),
            vmem_limit_bytes=56 * 1024 * 1024),
    )(x_flat, w_fc1.astype(jnp.float32), w_fc2.astype(jnp.float32), wcomb)
    return out_flat.reshape(N, C, H, W)


# speedup vs baseline: 1.0575x; 1.0575x over previous
import functools

import jax
import jax.numpy as jnp
from jax.experimental import pallas as pl
from jax.experimental.pallas import tpu as pltpu


def _cbam_body(x_ref, w1_ref, w2_ref, wc_ref, o_ref, *,
               inv_hw, inv_c, height, width, ksize):
    NB = x_ref.shape[0]
    H, W, k = height, width, ksize
    HW = H * W
    p = (k - 1) // 2
    f32 = jnp.float32

    xs = [x_ref[nb].astype(f32) for nb in range(NB)]
    ones_hw = jnp.ones((HW, 1), f32)
    cols = [jnp.dot(xc, ones_hw, preferred_element_type=f32) * inv_hw
            for xc in xs]
    cols += [jnp.max(xc, axis=-1, keepdims=True) for xc in xs]
    pooled = jnp.concatenate(cols, axis=1)
    h = jnp.dot(w1_ref[...], pooled, preferred_element_type=f32)
    h = jnp.maximum(h, 0.0)
    g = jnp.dot(w2_ref[...], h, preferred_element_type=f32)
    cgate = jax.nn.sigmoid(g[:, :NB] + g[:, NB:])

    C = x_ref.shape[1]
    ones_c = jnp.ones((1, C), f32)
    x1s = []
    mean_rows = []
    max_rows = []
    for nb in range(NB):
        x1 = xs[nb] * cgate[:, nb:nb + 1]
        x1s.append(x1)
        mean_rows.append(
            jnp.dot(ones_c, x1, preferred_element_type=f32) * inv_c)
        max_rows.append(jnp.max(x1, axis=0, keepdims=True))
    m2 = jnp.concatenate(mean_rows + max_rows, axis=0)

    lane = jax.lax.broadcasted_iota(jnp.int32, (1, HW), 1)
    orow = lane // W
    ocol = lane % W
    rowm = [((orow + (ki - p) >= 0) & (orow + (ki - p) < H)).astype(f32)
            for ki in range(k)]
    colm = [((ocol + (kj - p) >= 0) & (ocol + (kj - p) < W)).astype(f32)
            for kj in range(k)]
    rr = [pltpu.roll(m2, (-(ki - p) * W) % HW, axis=1) * rowm[ki]
          for ki in range(k)]
    conv = jnp.zeros((2 * NB, HW), f32)
    for kj in range(k):
        accs = [jnp.zeros((2 * NB, HW), f32) for _ in range(2)]
        for ki in range(k):
            wv = wc_ref[:, ki * k + kj:ki * k + kj + 1]
            accs[ki % 2] = accs[ki % 2] + rr[ki] * wv
        a = pltpu.roll(accs[0] + accs[1], (-(kj - p)) % HW, axis=1)
        conv = conv + a * colm[kj]
    sgate = jax.nn.sigmoid(conv[:NB] + conv[NB:])

    out_dt = o_ref.dtype
    for nb in range(NB):
        o_ref[nb] = (x1s[nb] * sgate[nb:nb + 1, :]).astype(out_dt)


def kernel(x, w_fc1, w_fc2, w_sp):
    N, C, H, W = x.shape
    Cr = w_fc1.shape[0]
    k = w_sp.shape[-1]
    HW = H * W

    NB = 4 if N % 4 == 0 else (2 if N % 2 == 0 else 1)

    x_flat = x.reshape(N, C, HW)
    wflat = w_sp.astype(jnp.float32).reshape(2, k * k)
    wcomb = jnp.concatenate(
        [jnp.tile(wflat[0:1], (NB, 1)), jnp.tile(wflat[1:2], (NB, 1))], axis=0)

    body = functools.partial(_cbam_body, inv_hw=1.0 / HW, inv_c=1.0 / C,
                             height=H, width=W, ksize=k)
    out_flat = pl.pallas_call(
        body,
        out_shape=jax.ShapeDtypeStruct((N, C, HW), x.dtype),
        grid=(N // NB,),
        in_specs=[
            pl.BlockSpec((NB, C, HW), lambda b: (b, 0, 0)),
            pl.BlockSpec((Cr, C), lambda b: (0, 0)),
            pl.BlockSpec((C, Cr), lambda b: (0, 0)),
            pl.BlockSpec((2 * NB, k * k), lambda b: (0, 0)),
        ],
        out_specs=pl.BlockSpec((NB, C, HW), lambda b: (b, 0, 0)),
        compiler_params=pltpu.CompilerParams(
            dimension_semantics=("parallel",),
            vmem_limit_bytes=56 * 1024 * 1024),
    )(x_flat, w_fc1.astype(jnp.float32), w_fc2.astype(jnp.float32), wcomb)
    return out_flat.reshape(N, C, H, W)


# speedup vs baseline: 1.0742x; 1.0157x over previous
import functools

import jax
import jax.numpy as jnp
from jax.experimental import pallas as pl
from jax.experimental.pallas import tpu as pltpu


def _cbam_body(x_ref, w1_ref, w2_ref, wc_ref, o_ref, *,
               inv_hw, inv_c, height, width, ksize):
    NB = x_ref.shape[0]
    H, W, k = height, width, ksize
    HW = H * W
    p = (k - 1) // 2
    f32 = jnp.float32

    xs = [x_ref[nb].astype(f32) for nb in range(NB)]
    cols = [jnp.sum(xc, axis=-1, keepdims=True) * inv_hw for xc in xs]
    cols += [jnp.max(xc, axis=-1, keepdims=True) for xc in xs]
    pooled = jnp.concatenate(cols, axis=1)
    h = jnp.dot(w1_ref[...], pooled, preferred_element_type=f32)
    h = jnp.maximum(h, 0.0)
    g = jnp.dot(w2_ref[...], h, preferred_element_type=f32)
    cgate = jax.nn.sigmoid(g[:, :NB] + g[:, NB:])

    x1s = []
    mean_rows = []
    max_rows = []
    for nb in range(NB):
        x1 = xs[nb] * cgate[:, nb:nb + 1]
        x1s.append(x1)
        mean_rows.append(jnp.sum(x1, axis=0, keepdims=True) * inv_c)
        max_rows.append(jnp.max(x1, axis=0, keepdims=True))
    m2 = jnp.concatenate(mean_rows + max_rows, axis=0)

    lane = jax.lax.broadcasted_iota(jnp.int32, (1, HW), 1)
    orow = lane // W
    ocol = lane % W
    rowm = [((orow + (ki - p) >= 0) & (orow + (ki - p) < H)).astype(f32)
            for ki in range(k)]
    colm = [((ocol + (kj - p) >= 0) & (ocol + (kj - p) < W)).astype(f32)
            for kj in range(k)]
    rr = [pltpu.roll(m2, (-(ki - p) * W) % HW, axis=1) * rowm[ki]
          for ki in range(k)]
    conv = jnp.zeros((2 * NB, HW), f32)
    for kj in range(k):
        accs = [jnp.zeros((2 * NB, HW), f32) for _ in range(2)]
        for ki in range(k):
            wv = wc_ref[:, ki * k + kj:ki * k + kj + 1]
            accs[ki % 2] = accs[ki % 2] + rr[ki] * wv
        a = pltpu.roll(accs[0] + accs[1], (-(kj - p)) % HW, axis=1)
        conv = conv + a * colm[kj]
    sgate = jax.nn.sigmoid(conv[:NB] + conv[NB:])

    out_dt = o_ref.dtype
    for nb in range(NB):
        o_ref[nb] = (x1s[nb] * sgate[nb:nb + 1, :]).astype(out_dt)


def kernel(x, w_fc1, w_fc2, w_sp):
    N, C, H, W = x.shape
    Cr = w_fc1.shape[0]
    k = w_sp.shape[-1]
    HW = H * W

    NB = 4 if N % 4 == 0 else (2 if N % 2 == 0 else 1)

    x_flat = x.reshape(N, C, HW)
    wflat = w_sp.astype(jnp.float32).reshape(2, k * k)
    wcomb = jnp.concatenate(
        [jnp.tile(wflat[0:1], (NB, 1)), jnp.tile(wflat[1:2], (NB, 1))], axis=0)

    body = functools.partial(_cbam_body, inv_hw=1.0 / HW, inv_c=1.0 / C,
                             height=H, width=W, ksize=k)
    out_flat = pl.pallas_call(
        body,
        out_shape=jax.ShapeDtypeStruct((N, C, HW), x.dtype),
        grid=(N // NB,),
        in_specs=[
            pl.BlockSpec((NB, C, HW), lambda b: (b, 0, 0)),
            pl.BlockSpec((Cr, C), lambda b: (0, 0)),
            pl.BlockSpec((C, Cr), lambda b: (0, 0)),
            pl.BlockSpec((2 * NB, k * k), lambda b: (0, 0)),
        ],
        out_specs=pl.BlockSpec((NB, C, HW), lambda b: (b, 0, 0)),
        compiler_params=pltpu.CompilerParams(
            dimension_semantics=("parallel",),
            vmem_limit_bytes=56 * 1024 * 1024),
    )(x_flat, w_fc1.astype(jnp.float32), w_fc2.astype(jnp.float32), wcomb)
    return out_flat.reshape(N, C, H, W)


# speedup vs baseline: 1.0890x; 1.0138x over previous
import functools

import jax
import jax.numpy as jnp
from jax.experimental import pallas as pl
from jax.experimental.pallas import tpu as pltpu


def _cbam_body(x_ref, w1_ref, w2_ref, wc_ref, o_ref, *,
               inv_hw, inv_c, height, width, ksize):
    NB = x_ref.shape[0]
    H, W, k = height, width, ksize
    HW = H * W
    p = (k - 1) // 2
    f32 = jnp.float32

    xs = [x_ref[nb].astype(f32) for nb in range(NB)]
    cols = [jnp.sum(xc, axis=-1, keepdims=True) * inv_hw for xc in xs]
    cols += [jnp.max(xc, axis=-1, keepdims=True) for xc in xs]
    pooled = jnp.concatenate(cols, axis=1)
    h = jnp.dot(w1_ref[...], pooled, preferred_element_type=f32)
    h = jnp.maximum(h, 0.0)
    g = jnp.dot(w2_ref[...], h, preferred_element_type=f32)
    cgate = jax.nn.sigmoid(g[:, :NB] + g[:, NB:])

    x1s = []
    mean_rows = []
    max_rows = []
    for nb in range(NB):
        x1 = xs[nb] * cgate[:, nb:nb + 1]
        x1s.append(x1)
        mean_rows.append(jnp.sum(x1, axis=0, keepdims=True) * inv_c)
        max_rows.append(jnp.max(x1, axis=0, keepdims=True))
    m2 = jnp.concatenate(mean_rows + max_rows, axis=0)

    lane = jax.lax.broadcasted_iota(jnp.int32, (1, HW), 1)
    orow = lane // W
    ocol = lane % W
    rowm = [((orow + (ki - p) >= 0) & (orow + (ki - p) < H)).astype(f32)
            for ki in range(k)]
    colm = [((ocol + (kj - p) >= 0) & (ocol + (kj - p) < W)).astype(f32)
            for kj in range(k)]
    accs = [jnp.zeros((2 * NB, HW), f32) for _ in range(2)]
    for ki in range(k):
        for kj in range(k):
            t = ki * k + kj
            s = (ki - p) * W + (kj - p)
            rolled = pltpu.roll(m2, (-s) % HW, axis=1)
            wv = wc_ref[:, t:t + 1]
            accs[t % 2] = accs[t % 2] + (rolled * wv) * (rowm[ki] * colm[kj])
    conv = accs[0] + accs[1]
    sgate = jax.nn.sigmoid(conv[:NB] + conv[NB:])

    out_dt = o_ref.dtype
    for nb in range(NB):
        o_ref[nb] = (x1s[nb] * sgate[nb:nb + 1, :]).astype(out_dt)


def kernel(x, w_fc1, w_fc2, w_sp):
    N, C, H, W = x.shape
    Cr = w_fc1.shape[0]
    k = w_sp.shape[-1]
    HW = H * W

    NB = 4 if N % 4 == 0 else (2 if N % 2 == 0 else 1)

    x_flat = x.reshape(N, C, HW)
    wflat = w_sp.astype(jnp.float32).reshape(2, k * k)
    wcomb = jnp.concatenate(
        [jnp.tile(wflat[0:1], (NB, 1)), jnp.tile(wflat[1:2], (NB, 1))], axis=0)

    body = functools.partial(_cbam_body, inv_hw=1.0 / HW, inv_c=1.0 / C,
                             height=H, width=W, ksize=k)
    out_flat = pl.pallas_call(
        body,
        out_shape=jax.ShapeDtypeStruct((N, C, HW), x.dtype),
        grid=(N // NB,),
        in_specs=[
            pl.BlockSpec((NB, C, HW), lambda b: (b, 0, 0)),
            pl.BlockSpec((Cr, C), lambda b: (0, 0)),
            pl.BlockSpec((C, Cr), lambda b: (0, 0)),
            pl.BlockSpec((2 * NB, k * k), lambda b: (0, 0)),
        ],
        out_specs=pl.BlockSpec((NB, C, HW), lambda b: (b, 0, 0)),
        compiler_params=pltpu.CompilerParams(
            dimension_semantics=("parallel",),
            vmem_limit_bytes=48 * 1024 * 1024),
    )(x_flat, w_fc1.astype(jnp.float32), w_fc2.astype(jnp.float32), wcomb)
    return out_flat.reshape(N, C, H, W)
